# Initial kernel scaffold; baseline (speedup 1.0000x reference)
import jax
import jax.numpy as jnp
from jax.experimental import pallas as pl


def kernel(x_in, table):
    raise NotImplementedError("write your pallas kernel here")


# speedup vs baseline: 3.3077x; 3.3077x over previous
import functools

import numpy as np
import jax
import jax.numpy as jnp
from jax import lax
from jax.experimental import pallas as pl
from jax.experimental.pallas import tpu as pltpu
from jax.experimental.pallas import tpu_sc as plsc

_D = 64
_S = 200
_CH = 100


def _pos_encoding_np(max_pos: int, d: int) -> np.ndarray:
    pos = np.arange(max_pos)[:, None].astype(np.float32)
    i = np.arange(d)[None, :]
    rates = 1.0 / np.power(10000.0, 2 * (i // 2) / np.float32(d))
    ang = pos * rates
    ang[:, 0::2] = np.sin(ang[:, 0::2])
    ang[:, 1::2] = np.cos(ang[:, 1::2])
    return ang.astype(np.float32)


_PE = _pos_encoding_np(256, _D)[:_S]


def kernel(x_in, table):
    B, S = x_in.shape
    D = table.shape[1]
    N = B * S
    info = plsc.get_sparse_core_info()
    NC, NS = info.num_cores, info.num_subcores
    NW = NC * NS
    n_per_w = N // NW
    seq_per_w = n_per_w // S
    chunks = n_per_w // _CH

    x3 = x_in.astype(jnp.int32).reshape(NW, chunks, _CH)
    pe = jnp.asarray(_PE)

    mesh = plsc.VectorSubcoreMesh(core_axis_name="c", subcore_axis_name="s")

    @functools.partial(
        pl.kernel,
        mesh=mesh,
        out_type=jax.ShapeDtypeStruct((N, D), jnp.float32),
        scratch_types=[
            pltpu.VMEM((chunks, _CH), jnp.int32),
            pltpu.VMEM((S, D), jnp.float32),
            pltpu.VMEM((S, D), jnp.float32),
            pltpu.SemaphoreType.DMA,
        ],
        compiler_params=pltpu.CompilerParams(use_tc_tiling_on_sc=False),
    )
    def run(x_hbm, table_hbm, pe_hbm, out_hbm, idx_v, pe_v, buf_v, sem):
        wid = lax.axis_index("s") * NC + lax.axis_index("c")
        pltpu.sync_copy(pe_hbm, pe_v)
        pltpu.sync_copy(x_hbm.at[wid], idx_v)

        def seq_body(j, carry):
            c0 = 2 * j
            cp0 = pltpu.async_copy(
                table_hbm.at[idx_v.at[c0]], buf_v.at[pl.ds(0, _CH)], sem)
            cp1 = pltpu.async_copy(
                table_hbm.at[idx_v.at[c0 + 1]], buf_v.at[pl.ds(_CH, _CH)], sem)
            cp0.wait()
            cp1.wait()

            def add_body(r, c):
                for k in range(D // 16):
                    sl = pl.ds(k * 16, 16)
                    buf_v[r, sl] = buf_v[r, sl] + pe_v[r, sl]
                return c

            lax.fori_loop(0, S, add_body, None)
            base = (wid * seq_per_w + j) * S
            pltpu.sync_copy(buf_v, out_hbm.at[pl.ds(base, S)])
            return carry

        lax.fori_loop(0, seq_per_w, seq_body, None)

    out = run(x3, table, pe)
    return out.reshape(B, S, D)


# speedup vs baseline: 3.4277x; 1.0363x over previous
import functools

import numpy as np
import jax
import jax.numpy as jnp
from jax import lax
from jax.experimental import pallas as pl
from jax.experimental.pallas import tpu as pltpu
from jax.experimental.pallas import tpu_sc as plsc

_D = 64
_S = 200
_CH = 100
_NBUF = 4


def _pos_encoding_np(max_pos: int, d: int) -> np.ndarray:
    pos = np.arange(max_pos)[:, None].astype(np.float32)
    i = np.arange(d)[None, :]
    rates = 1.0 / np.power(10000.0, 2 * (i // 2) / np.float32(d))
    ang = pos * rates
    ang[:, 0::2] = np.sin(ang[:, 0::2])
    ang[:, 1::2] = np.cos(ang[:, 1::2])
    return ang.astype(np.float32)


_PE = _pos_encoding_np(256, _D)[:_S]


def kernel(x_in, table):
    B, S = x_in.shape
    D = table.shape[1]
    N = B * S
    info = plsc.get_sparse_core_info()
    NC, NS = info.num_cores, info.num_subcores
    NW = NC * NS
    n_per_w = N // NW
    seq_per_w = n_per_w // S
    chunks = n_per_w // _CH

    x3 = x_in.astype(jnp.int32).reshape(NW, chunks, _CH)
    pe = jnp.asarray(_PE)

    mesh = plsc.VectorSubcoreMesh(core_axis_name="c", subcore_axis_name="s")

    @functools.partial(
        pl.kernel,
        mesh=mesh,
        out_type=jax.ShapeDtypeStruct((N, D), jnp.float32),
        scratch_types=[
            pltpu.VMEM((chunks, _CH), jnp.int32),
            pltpu.VMEM((S, D), jnp.float32),
            pltpu.VMEM((_NBUF, S, D), jnp.float32),
        ] + [pltpu.SemaphoreType.DMA] * (2 * _NBUF),
        compiler_params=pltpu.CompilerParams(use_tc_tiling_on_sc=False),
    )
    def run(x_hbm, table_hbm, pe_hbm, out_hbm, idx_v, pe_v, buf_v, *sems):
        gsem = sems[:_NBUF]
        osem = sems[_NBUF:]
        wid = lax.axis_index("s") * NC + lax.axis_index("c")
        pltpu.sync_copy(pe_hbm, pe_v)
        pltpu.sync_copy(x_hbm.at[wid], idx_v)
        row0 = wid * seq_per_w

        def issue_gather(j, b):
            c0 = 2 * j
            pltpu.async_copy(
                table_hbm.at[idx_v.at[c0]], buf_v.at[b, pl.ds(0, _CH)],
                gsem[b])
            pltpu.async_copy(
                table_hbm.at[idx_v.at[c0 + 1]], buf_v.at[b, pl.ds(_CH, _CH)],
                gsem[b])

        def drain(sem, b):
            pltpu.make_async_copy(
                out_hbm.at[pl.ds(0, S)], buf_v.at[b], sem).wait()

        issue_gather(0, 0)
        issue_gather(1, 1)

        @pl.loop(0, seq_per_w, step=_NBUF)
        def _(jj):
            for b in range(_NBUF):
                j = jj + b
                b2 = (b + 2) % _NBUF

                drain(gsem[b], b)

                @pl.loop(0, S, step=4, unroll=2)
                def _(r0):
                    for dr in range(4):
                        r = r0 + dr
                        for k in range(D // 16):
                            sl = pl.ds(k * 16, 16)
                            buf_v[b, r, sl] = buf_v[b, r, sl] + pe_v[r, sl]

                pltpu.async_copy(
                    buf_v.at[b], out_hbm.at[pl.ds((row0 + j) * S, S)],
                    osem[b])

                jn = j + 2

                @pl.when(jn < seq_per_w)
                def _():
                    @pl.when(jn >= _NBUF)
                    def _():
                        drain(osem[b2], b2)
                    issue_gather(jn, b2)

        for b in range(_NBUF):
            drain(osem[b], b)

    out = run(x3, table, pe)
    return out.reshape(B, S, D)
